# Initial kernel scaffold; baseline (speedup 1.0000x reference)
import jax
import jax.numpy as jnp
from jax.experimental import pallas as pl


def kernel(x_cat, x_cont, emb_tables, W1, b1, W2, b2, bn1_gamma, bn1_beta, bn1_mean, bn1_var, bn2_gamma, bn2_beta, bn2_mean, bn2_var):
    raise NotImplementedError("write your pallas kernel here")


# speedup vs baseline: 47.4909x; 47.4909x over previous
import functools

import jax
import jax.numpy as jnp
from jax import lax
from jax.experimental import pallas as pl
from jax.experimental.pallas import tpu as pltpu
from jax.experimental.pallas import tpu_sc as plsc

B = 16384
F = 26
V = 100000
D = 16
K = F * D
N_CONT = 13
H = 200
OUT = 5
EPS = 1e-5

NW = 32
RPW = K // NW
NCH = 4
CHS = B // NCH // 128


def _sc_gather_body(tbl, xct, out, trow, idxb, ob0, ob1, sem_r, sem_i, sem_w):
    wid = lax.axis_index("s") * 2 + lax.axis_index("c")
    lane = lax.iota(jnp.int32, 16)

    def row_body(j, carry):
        k = wid * RPW + j
        f = k // D
        d = lax.rem(k, D)

        @pl.when(jnp.logical_or(j == 0, d == 0))
        def _load_idx():
            pltpu.async_copy(xct.at[f], idxb, sem_i).wait()

        cp_r = pltpu.async_copy(tbl.at[f, d], trow, sem_r)
        cp_r.wait()
        wbs = []
        for c in range(NCH):
            ob = ob0 if c % 2 == 0 else ob1
            if c >= 2:
                wbs[c - 2].wait()

            def g_body(r, carry2, c=c, ob=ob):
                for u in range(8):
                    b0 = c * (CHS * 128) + r * 128 + u * 16
                    idxv = idxb[pl.ds(b0, 16)]
                    vals = plsc.load_gather(trow, [idxv])
                    ob[r, pl.ds(u * 16, 16)] = vals
                return carry2

            lax.fori_loop(0, CHS, g_body, 0)
            wbs.append(
                pltpu.async_copy(ob, out.at[k, pl.ds(c * CHS, CHS)], sem_w)
            )
        wbs[NCH - 2].wait()
        wbs[NCH - 1].wait()
        return carry

    lax.fori_loop(0, RPW, row_body, 0)
    del lane


@functools.cache
def _make_sc_gather():
    mesh = plsc.VectorSubcoreMesh(
        core_axis_name="c", subcore_axis_name="s", num_cores=2, num_subcores=16
    )
    return pl.kernel(
        _sc_gather_body,
        out_type=jax.ShapeDtypeStruct((K, B // 128, 128), jnp.float32),
        mesh=mesh,
        scratch_types=[
            pltpu.VMEM((V,), jnp.float32),
            pltpu.VMEM((B,), jnp.int32),
            pltpu.VMEM((CHS, 128), jnp.float32),
            pltpu.VMEM((CHS, 128), jnp.float32),
            pltpu.SemaphoreType.DMA,
            pltpu.SemaphoreType.DMA,
            pltpu.SemaphoreType.DMA,
        ],
        compiler_params=pltpu.CompilerParams(use_tc_tiling_on_sc=True, needs_layout_passes=False),
    )


BBH = 8


def _mlp_body(gt, xc, w1a, w1b, b1, w2, b2, o):
    dn = (((0,), (0,)), ((), ()))
    for s in range(BBH):
        gs = gt[:, s, :]
        h = lax.dot_general(gs, w1a[...], dn, preferred_element_type=jnp.float32)
        h = h + lax.dot_general(xc[:, s * 128:(s + 1) * 128], w1b[...], dn,
                                preferred_element_type=jnp.float32)
        h = jnp.maximum(h + b1[...], 0.0)
        o[pl.ds(s * 128, 128), :] = (
            jnp.dot(h, w2[...], preferred_element_type=jnp.float32) + b2[...]
        )


def _full(r, c):
    return pl.BlockSpec((r, c), lambda i: (0, 0))


_mlp_call = pl.pallas_call(
    _mlp_body,
    grid=(B // (BBH * 128),),
    in_specs=[
        pl.BlockSpec((K, BBH, 128), lambda i: (0, i, 0)),
        pl.BlockSpec((N_CONT, BBH * 128), lambda i: (0, i)),
        _full(K, H),
        _full(N_CONT, H),
        _full(1, H),
        _full(H, OUT),
        _full(1, OUT),
    ],
    out_specs=pl.BlockSpec((BBH * 128, OUT), lambda i: (i, 0)),
    out_shape=jax.ShapeDtypeStruct((B, OUT), jnp.float32),
    compiler_params=pltpu.CompilerParams(
        dimension_semantics=("arbitrary",),
    ),
)


def kernel(x_cat, x_cont, emb_tables, W1, b1, W2, b2,
           bn1_gamma, bn1_beta, bn1_mean, bn1_var,
           bn2_gamma, bn2_beta, bn2_mean, bn2_var):
    tbl = jnp.transpose(emb_tables, (0, 2, 1))
    xct = jnp.transpose(x_cat.astype(jnp.int32), (1, 0))
    gt = _make_sc_gather()(tbl, xct)
    xc_t = jnp.transpose(x_cont, (1, 0))
    s1 = bn1_gamma * lax.rsqrt(bn1_var + EPS)
    w1b = W1[K:] * s1[:, None]
    b1f = b1 + (bn1_beta - bn1_mean * s1) @ W1[K:]
    s2 = bn2_gamma * lax.rsqrt(bn2_var + EPS)
    w2f = W2 * s2[:, None]
    b2f = b2 + (bn2_beta - bn2_mean * s2) @ W2
    out = _mlp_call(
        gt, xc_t, W1[:K], w1b, b1f.reshape(1, H), w2f, b2f.reshape(1, OUT),
    )
    return out


# speedup vs baseline: 47.8812x; 1.0082x over previous
import functools

import jax
import jax.numpy as jnp
from jax import lax
from jax.experimental import pallas as pl
from jax.experimental.pallas import tpu as pltpu
from jax.experimental.pallas import tpu_sc as plsc

B = 16384
F = 26
V = 100000
D = 16
K = F * D
N_CONT = 13
H = 200
OUT = 5
EPS = 1e-5

NW = 32
RPW = K // NW
NCH = 4
CHS = B // NCH // 128


def _sc_gather_body(k0, rpw, tbl, xct, out, trow, idxb, ob0, ob1,
                    sem_r, sem_i, sem_w):
    wid = lax.axis_index("s") * 2 + lax.axis_index("c")
    lane = lax.iota(jnp.int32, 16)

    def row_body(j, carry):
        k = k0 + wid * rpw + j
        f = k // D
        d = lax.rem(k, D)

        @pl.when(jnp.logical_or(j == 0, d == 0))
        def _load_idx():
            pltpu.async_copy(xct.at[f], idxb, sem_i).wait()

        cp_r = pltpu.async_copy(tbl.at[f, d], trow, sem_r)
        cp_r.wait()
        wbs = []
        for c in range(NCH):
            ob = ob0 if c % 2 == 0 else ob1
            if c >= 2:
                wbs[c - 2].wait()

            def g_body(r, carry2, c=c, ob=ob):
                for u in range(8):
                    b0 = c * (CHS * 128) + r * 128 + u * 16
                    idxv = idxb[pl.ds(b0, 16)]
                    vals = plsc.load_gather(trow, [idxv])
                    ob[r, pl.ds(u * 16, 16)] = vals
                return carry2

            lax.fori_loop(0, CHS, g_body, 0)
            wbs.append(
                pltpu.async_copy(ob, out.at[k - k0, pl.ds(c * CHS, CHS)], sem_w)
            )
        wbs[NCH - 2].wait()
        wbs[NCH - 1].wait()
        return carry

    lax.fori_loop(0, rpw, row_body, 0)
    del lane


@functools.cache
def _make_sc_gather(k0, nrows):
    mesh = plsc.VectorSubcoreMesh(
        core_axis_name="c", subcore_axis_name="s", num_cores=2, num_subcores=16
    )
    return pl.kernel(
        functools.partial(_sc_gather_body, k0, nrows // NW),
        out_type=jax.ShapeDtypeStruct((nrows, B // 128, 128), jnp.float32),
        mesh=mesh,
        scratch_types=[
            pltpu.VMEM((V,), jnp.float32),
            pltpu.VMEM((B,), jnp.int32),
            pltpu.VMEM((CHS, 128), jnp.float32),
            pltpu.VMEM((CHS, 128), jnp.float32),
            pltpu.SemaphoreType.DMA,
            pltpu.SemaphoreType.DMA,
            pltpu.SemaphoreType.DMA,
        ],
        compiler_params=pltpu.CompilerParams(use_tc_tiling_on_sc=True, needs_layout_passes=False),
    )


BBH = 8
K1 = 256
K2 = K - K1


def _mlp1_body(gt, xc, w1a, w1b, b1, o):
    dn = (((0,), (0,)), ((), ()))
    for s in range(BBH):
        h = lax.dot_general(gt[:, s, :], w1a[...], dn,
                            preferred_element_type=jnp.float32)
        h = h + lax.dot_general(xc[:, s * 128:(s + 1) * 128], w1b[...], dn,
                                preferred_element_type=jnp.float32)
        o[pl.ds(s * 128, 128), :] = h + b1[...]


def _mlp2_body(gt, p, w1a, w2, b2, o):
    dn = (((0,), (0,)), ((), ()))
    for s in range(BBH):
        h = p[pl.ds(s * 128, 128), :] + lax.dot_general(
            gt[:, s, :], w1a[...], dn, preferred_element_type=jnp.float32)
        h = jnp.maximum(h, 0.0)
        o[pl.ds(s * 128, 128), :] = (
            jnp.dot(h, w2[...], preferred_element_type=jnp.float32) + b2[...]
        )


def _full(r, c):
    return pl.BlockSpec((r, c), lambda i: (0, 0))


_mlp1_call = pl.pallas_call(
    _mlp1_body,
    grid=(B // (BBH * 128),),
    in_specs=[
        pl.BlockSpec((K1, BBH, 128), lambda i: (0, i, 0)),
        pl.BlockSpec((N_CONT, BBH * 128), lambda i: (0, i)),
        _full(K1, H),
        _full(N_CONT, H),
        _full(1, H),
    ],
    out_specs=pl.BlockSpec((BBH * 128, H), lambda i: (i, 0)),
    out_shape=jax.ShapeDtypeStruct((B, H), jnp.float32),
    compiler_params=pltpu.CompilerParams(
        dimension_semantics=("arbitrary",),
    ),
)

_mlp2_call = pl.pallas_call(
    _mlp2_body,
    grid=(B // (BBH * 128),),
    in_specs=[
        pl.BlockSpec((K2, BBH, 128), lambda i: (0, i, 0)),
        pl.BlockSpec((BBH * 128, H), lambda i: (i, 0)),
        _full(K2, H),
        _full(H, OUT),
        _full(1, OUT),
    ],
    out_specs=pl.BlockSpec((BBH * 128, OUT), lambda i: (i, 0)),
    out_shape=jax.ShapeDtypeStruct((B, OUT), jnp.float32),
    compiler_params=pltpu.CompilerParams(
        dimension_semantics=("arbitrary",),
    ),
)


def kernel(x_cat, x_cont, emb_tables, W1, b1, W2, b2,
           bn1_gamma, bn1_beta, bn1_mean, bn1_var,
           bn2_gamma, bn2_beta, bn2_mean, bn2_var):
    tbl = jnp.transpose(emb_tables, (0, 2, 1))
    xct = jnp.transpose(x_cat.astype(jnp.int32), (1, 0))
    gt1 = _make_sc_gather(0, K1)(tbl, xct)
    gt2 = _make_sc_gather(K1, K2)(tbl, xct)
    xc_t = jnp.transpose(x_cont, (1, 0))
    s1 = bn1_gamma * lax.rsqrt(bn1_var + EPS)
    w1b = W1[K:] * s1[:, None]
    b1f = b1 + (bn1_beta - bn1_mean * s1) @ W1[K:]
    s2 = bn2_gamma * lax.rsqrt(bn2_var + EPS)
    w2f = W2 * s2[:, None]
    b2f = b2 + (bn2_beta - bn2_mean * s2) @ W2
    p = _mlp1_call(gt1, xc_t, W1[:K1], w1b, b1f.reshape(1, H))
    out = _mlp2_call(gt2, p, W1[K1:K], w2f, b2f.reshape(1, OUT))
    return out
